# baseline (speedup 1.0000x reference)
import functools

import jax
import jax.numpy as jnp
from jax import lax
from jax.experimental import pallas as pl
from jax.experimental.pallas import tpu as pltpu
from jax.experimental.pallas import tpu_sc as plsc

B, N, V = 8, 4096, 1024
NC, NS, L = 2, 16, 16
WPB = 4
Q = N // WPB
CH = 128
NCHUNK = Q // CH
NBUF = 4


def _body(rows_hbm, idvec_hbm, out_hbm, idvec_v, vals0, vals1, vals2, vals3,
          loc_v, idx_v, acc_sm, out_sh, sem0, sem1, sem2, sem3, sem4):
    c = lax.axis_index("c")
    s = lax.axis_index("s")
    batch = c * (NS // WPB) + (s >> 2)
    bloc = s >> 2
    q = s & (WPB - 1)
    row_base = batch * N + q * Q

    acc_sm[0] = 0

    pltpu.sync_copy(idvec_hbm, idvec_v)
    idv = idvec_v[...]
    id_lane = idv & 127
    col0 = pl.multiple_of((idv[0] >> 7) << 7, 128)
    lane = lax.broadcasted_iota(jnp.int32, (L,), 0)
    zero = jnp.zeros((L,), jnp.int32)

    def src(k):
        return rows_hbm.at[pl.ds(row_base + k * CH, CH), pl.ds(col0, 128)]

    bufs = [vals0, vals1, vals2, vals3]
    sems = [sem0, sem1, sem2, sem3]
    for u in range(NBUF):
        pltpu.async_copy(src(u), bufs[u], sems[u])

    def zfill(j, carry):
        loc_v[pl.ds(j * L, L)] = zero
        return carry

    lax.fori_loop(0, Q // L, zfill, 0)

    def compact_chunk(vals_ref, k, cnt):
        def step(t2, cnt):
            for u in range(2):
                r = (t2 * 2 + u) * L + lane
                v = plsc.load_gather(vals_ref, [r, id_lane])
                m = v != 0.0
                csum = plsc.cumsum(m.astype(jnp.int32))
                pos = cnt + csum - 1
                plsc.store_scatter(loc_v, [pos], q * Q + k * CH + r, mask=m)
                cnt = cnt + plsc.all_reduce_population_count(m)
            return cnt

        return lax.fori_loop(0, CH // L // 2, step, cnt)

    def outer(g, cnt):
        for u in range(NBUF):
            k = NBUF * g + u
            pltpu.make_async_copy(src(k), bufs[u], sems[u]).wait()
            cnt = compact_chunk(bufs[u], k, cnt)

            @pl.when(k + NBUF < NCHUNK)
            def _():
                pltpu.async_copy(src(k + NBUF), bufs[u], sems[u])

        return cnt

    cnt = lax.fori_loop(0, NCHUNK // NBUF, outer, jnp.zeros((L,), jnp.int32))
    cnt_s = cnt[0]

    init = jnp.where(q == 0, 1, 0)

    def poll_cond(carry):
        v, i = carry
        return (v == 0) & (i < (1 << 20))

    def poll_body(carry):
        v, i = carry
        return plsc.fetch_and_add(acc_sm, 0, subcore_id=s), i + 1

    v, _ = lax.while_loop(poll_cond, poll_body, (init, jnp.int32(0)))
    prefix_s = v - 1

    @pl.when(q < WPB - 1)
    def _():
        plsc.fetch_and_add(acc_sm, prefix_s + cnt_s + 1, subcore_id=s + 1)

    spares = q * Q - prefix_s
    front0 = bloc * N + prefix_s
    tail0 = bloc * N + (N - 1) - spares + cnt

    def ifill(t, carry):
        jvec = t * L + lane
        idx_v[pl.ds(t * L, L)] = jnp.where(
            jvec < cnt, front0 + jvec, tail0 - jvec
        )
        return carry

    lax.fori_loop(0, Q // L, ifill, 0)
    pltpu.async_copy(loc_v, out_sh.at[idx_v], sem4).wait()
    plsc.subcore_barrier()

    pltpu.sync_copy(
        out_sh.at[pl.ds(bloc * N + q * Q, Q)],
        out_hbm.at[batch, pl.ds(q * Q, Q)],
    )


@functools.partial(
    pl.kernel,
    out_type=jax.ShapeDtypeStruct((B, N), jnp.int32),
    mesh=plsc.VectorSubcoreMesh(core_axis_name="c", subcore_axis_name="s"),
    scratch_types=[
        pltpu.VMEM((L,), jnp.int32),
        pltpu.VMEM((CH, 128), jnp.float32),
        pltpu.VMEM((CH, 128), jnp.float32),
        pltpu.VMEM((CH, 128), jnp.float32),
        pltpu.VMEM((CH, 128), jnp.float32),
        pltpu.VMEM((Q,), jnp.int32),
        pltpu.VMEM((Q,), jnp.int32),
        pltpu.SMEM((1,), jnp.int32),
        pltpu.VMEM_SHARED((N * NS // WPB,), jnp.int32),
        pltpu.SemaphoreType.DMA,
        pltpu.SemaphoreType.DMA,
        pltpu.SemaphoreType.DMA,
        pltpu.SemaphoreType.DMA,
        pltpu.SemaphoreType.DMA,
    ],
    compiler_params=pltpu.CompilerParams(needs_layout_passes=False),
)
def _filter_sc(rows_hbm, idvec_hbm, out_hbm, idvec_v, vals0, vals1, vals2,
               vals3, loc_v, idx_v, acc_sm, out_sh, sem0, sem1, sem2, sem3,
               sem4):
    _body(rows_hbm, idvec_hbm, out_hbm, idvec_v, vals0, vals1, vals2, vals3,
          loc_v, idx_v, acc_sm, out_sh, sem0, sem1, sem2, sem3, sem4)


def kernel(one_hot, id):
    rows = one_hot.reshape(B * N, V)
    idvec = jnp.full((L,), id, dtype=jnp.int32)
    return _filter_sc(rows, idvec)


# speedup vs baseline: 1.0074x; 1.0074x over previous
import functools

import jax
import jax.numpy as jnp
from jax import lax
from jax.experimental import pallas as pl
from jax.experimental.pallas import tpu as pltpu
from jax.experimental.pallas import tpu_sc as plsc

B, N, V = 8, 4096, 1024
NC, NS, L = 2, 16, 16
WPB = 4
Q = N // WPB
CH = 256
NCHUNK = Q // CH


def _body(rows_hbm, idvec_hbm, out_hbm, idvec_v, vals0, vals1, loc_v, idx_v,
          acc_sm, out_sh, sem0, sem1, sem2):
    c = lax.axis_index("c")
    s = lax.axis_index("s")
    batch = c * (NS // WPB) + (s >> 2)
    bloc = s >> 2
    q = s & (WPB - 1)
    row_base = batch * N + q * Q

    acc_sm[0] = 0

    pltpu.sync_copy(idvec_hbm, idvec_v)
    idv = idvec_v[...]
    id_lane = idv & 127
    col0 = pl.multiple_of((idv[0] >> 7) << 7, 128)
    lane = lax.broadcasted_iota(jnp.int32, (L,), 0)
    zero = jnp.zeros((L,), jnp.int32)

    def src(k):
        return rows_hbm.at[pl.ds(row_base + k * CH, CH), pl.ds(col0, 128)]

    pltpu.async_copy(src(0), vals0, sem0)
    pltpu.async_copy(src(1), vals1, sem1)

    def zfill(j, carry):
        loc_v[pl.ds(j * L, L)] = zero
        return carry

    lax.fori_loop(0, Q // L, zfill, 0)

    def compact_chunk(vals_ref, k, cnt):
        def step(t, cnt):
            r = t * L + lane
            v = plsc.load_gather(vals_ref, [r, id_lane])
            m = v != 0.0
            csum = plsc.cumsum(m.astype(jnp.int32))
            pos = cnt + csum - 1
            plsc.store_scatter(loc_v, [pos], q * Q + k * CH + r, mask=m)
            return cnt + plsc.all_reduce_population_count(m)

        return lax.fori_loop(0, CH // L, step, cnt)

    def outer(g, cnt):
        k0 = 2 * g
        pltpu.make_async_copy(src(k0), vals0, sem0).wait()
        cnt = compact_chunk(vals0, k0, cnt)

        @pl.when(k0 + 2 < NCHUNK)
        def _():
            pltpu.async_copy(src(k0 + 2), vals0, sem0)

        k1 = 2 * g + 1
        pltpu.make_async_copy(src(k1), vals1, sem1).wait()
        cnt = compact_chunk(vals1, k1, cnt)

        @pl.when(k1 + 2 < NCHUNK)
        def _():
            pltpu.async_copy(src(k1 + 2), vals1, sem1)

        return cnt

    cnt = lax.fori_loop(0, NCHUNK // 2, outer, jnp.zeros((L,), jnp.int32))
    cnt_s = cnt[0]

    init = jnp.where(q == 0, 1, 0)

    def poll_cond(carry):
        v, i = carry
        return (v == 0) & (i < (1 << 20))

    def poll_body(carry):
        v, i = carry
        return plsc.fetch_and_add(acc_sm, 0, subcore_id=s), i + 1

    v, _ = lax.while_loop(poll_cond, poll_body, (init, jnp.int32(0)))
    prefix_s = v - 1

    @pl.when(q < WPB - 1)
    def _():
        plsc.fetch_and_add(acc_sm, prefix_s + cnt_s + 1, subcore_id=s + 1)

    spares = q * Q - prefix_s
    front0 = bloc * N + prefix_s
    tail0 = bloc * N + (N - 1) - spares + cnt

    def ifill(t, carry):
        jvec = t * L + lane
        idx_v[pl.ds(t * L, L)] = jnp.where(
            jvec < cnt, front0 + jvec, tail0 - jvec
        )
        return carry

    lax.fori_loop(0, Q // L, ifill, 0)
    pltpu.async_copy(loc_v, out_sh.at[idx_v], sem2).wait()
    plsc.subcore_barrier()

    @pl.when(q == 0)
    def _():
        pltpu.sync_copy(out_sh.at[pl.ds(bloc * N, N)], out_hbm.at[batch])


@functools.partial(
    pl.kernel,
    out_type=jax.ShapeDtypeStruct((B, N), jnp.int32),
    mesh=plsc.VectorSubcoreMesh(core_axis_name="c", subcore_axis_name="s"),
    scratch_types=[
        pltpu.VMEM((L,), jnp.int32),
        pltpu.VMEM((CH, 128), jnp.float32),
        pltpu.VMEM((CH, 128), jnp.float32),
        pltpu.VMEM((Q,), jnp.int32),
        pltpu.VMEM((Q,), jnp.int32),
        pltpu.SMEM((1,), jnp.int32),
        pltpu.VMEM_SHARED((N * NS // WPB,), jnp.int32),
        pltpu.SemaphoreType.DMA,
        pltpu.SemaphoreType.DMA,
        pltpu.SemaphoreType.DMA,
    ],
    compiler_params=pltpu.CompilerParams(needs_layout_passes=False),
)
def _filter_sc(rows_hbm, idvec_hbm, out_hbm, idvec_v, vals0, vals1, loc_v,
               idx_v, acc_sm, out_sh, sem0, sem1, sem2):
    _body(rows_hbm, idvec_hbm, out_hbm, idvec_v, vals0, vals1, loc_v, idx_v,
          acc_sm, out_sh, sem0, sem1, sem2)


def kernel(one_hot, id):
    rows = one_hot.reshape(B * N, V)
    idvec = jnp.full((L,), id, dtype=jnp.int32)
    return _filter_sc(rows, idvec)
